# baseline (speedup 1.0000x reference)
import functools

import jax
import jax.numpy as jnp
from jax import lax
from jax.experimental import pallas as pl
from jax.experimental.pallas import tpu as pltpu
from jax.experimental.pallas import tpu_sc as plsc

N = 10000
NP = 10240
K = 6
C = 32
RB = 512
MB = 2560
NW = 32
NC = 2
NPW = NP // NW
EPW = NPW * K
NPCH = 16
ECH = NPCH * K
NCH = EPW // ECH
NSLOT = 20
CW = 32
BIG = 2 ** 30


CBW = 512
NCB = NP // CBW
HUGE = 1e30


def _knn_body(clo_ref, chi_ref, pos_blk, batch_blk, postc_ref, batchrc_ref,
              nbrs_ref):
    b = pl.program_id(0)
    c_lo = clo_ref[b]
    c_hi = chi_ref[b]
    pe = pos_blk[...]
    be = batch_blk[...]
    inf = jnp.float32(jnp.inf)
    rv0 = jnp.full((RB, 8), inf, jnp.float32)
    ri0 = jnp.full((RB, 8), BIG, jnp.int32)

    def chunk(cb, carry):
        rv, ri = carry
        pt = postc_ref[cb]
        br = batchrc_ref[cb]
        d2 = None
        for c in range(3):
            diff = pe[:, c:c + 1] - pt[c:c + 1, :]
            sq = diff * diff
            d2 = sq if d2 is None else d2 + sq
        d2 = jnp.where(be != br, jnp.float32(HUGE), d2)
        idxc = lax.broadcasted_iota(jnp.int32, (RB, CBW), 1) + cb * CBW
        full = jnp.concatenate([d2, rv], axis=1)
        fidx = jnp.concatenate([idxc, ri], axis=1)
        nrv, nri = [], []
        for _ in range(K):
            m = jnp.min(full, axis=1, keepdims=True)
            sel = jnp.where(full == m, fidx, BIG)
            j = jnp.min(sel, axis=1, keepdims=True)
            nrv.append(m)
            nri.append(j)
            full = jnp.where(fidx == j, inf, full)
        pad_v = jnp.full((RB, 8 - K), inf, jnp.float32)
        pad_i = jnp.full((RB, 8 - K), BIG, jnp.int32)
        return (jnp.concatenate(nrv + [pad_v], axis=1),
                jnp.concatenate(nri + [pad_i], axis=1))

    rv, ri = lax.fori_loop(c_lo, c_hi, chunk, (rv0, ri0))
    nbrs_ref[...] = jnp.minimum(ri[:, :K], NP - 1)


def _knn(pos_p, batch_col, pos_tc, batch_rc, c_lo, c_hi):
    return pl.pallas_call(
        _knn_body,
        grid=(NP // RB,),
        in_specs=[
            pl.BlockSpec(memory_space=pltpu.SMEM),
            pl.BlockSpec(memory_space=pltpu.SMEM),
            pl.BlockSpec((RB, 3), lambda i: (i, 0)),
            pl.BlockSpec((RB, 1), lambda i: (i, 0)),
            pl.BlockSpec((NCB, 3, CBW), lambda i: (0, 0, 0)),
            pl.BlockSpec((NCB, 1, CBW), lambda i: (0, 0, 0)),
        ],
        out_specs=pl.BlockSpec((RB, K), lambda i: (i, 0)),
        out_shape=jax.ShapeDtypeStruct((NP, K), jnp.int32),
    )(c_lo, c_hi, pos_p, batch_col, pos_tc, batch_rc)


def _mlp_body(h_ref, wa_ref, ba_ref, wb_ref, bb_ref, u_ref):
    a = jnp.dot(h_ref[...], wa_ref[...], preferred_element_type=jnp.float32)
    a = jnp.maximum(a + ba_ref[...], 0.0)
    u_ref[...] = (jnp.dot(a, wb_ref[...], preferred_element_type=jnp.float32)
                  + bb_ref[...])


def _mlp(h, Wa, ba, Wb, bb):
    cin = h.shape[1]
    return pl.pallas_call(
        _mlp_body,
        grid=(NP // MB,),
        in_specs=[
            pl.BlockSpec((MB, cin), lambda i: (i, 0)),
            pl.BlockSpec((cin, C), lambda i: (0, 0)),
            pl.BlockSpec((1, C), lambda i: (0, 0)),
            pl.BlockSpec((C, C), lambda i: (0, 0)),
            pl.BlockSpec((1, C), lambda i: (0, 0)),
        ],
        out_specs=pl.BlockSpec((MB, C), lambda i: (i, 0)),
        out_shape=jax.ShapeDtypeStruct((NP, C), jnp.float32),
    )(h, Wa, ba, Wb, bb)


def _gather_max_sc_body(u_hbm, e_hbm, out_hbm, idx_v, rows_v, out_v, sem):
    w = lax.axis_index("s") * NC + lax.axis_index("c")
    pltpu.sync_copy(e_hbm.at[w], idx_v)

    def fire(g, slot):
        return pltpu.async_copy(u_hbm.at[idx_v.at[g]], rows_v.at[slot], sem)

    cps = {}
    for g in range(min(NSLOT, NCH)):
        cps[g] = fire(g, g)
    for g in range(NCH):
        cps.pop(g).wait()
        slot = g % NSLOT

        def body(i, slot=slot, g=g):
            base = i * K
            for c2 in range(C // 16):
                v = rows_v[slot, base, pl.ds(c2 * 16, 16)]
                for k in range(1, K):
                    v = jnp.maximum(v, rows_v[slot, base + k,
                                              pl.ds(c2 * 16, 16)])
                out_v[g * NPCH + i, pl.ds(c2 * 16, 16)] = jnp.maximum(v, 0.0)

        plsc.parallel_loop(0, NPCH, unroll=4)(body)
        nxt = g + NSLOT
        if nxt < NCH:
            cps[nxt] = fire(nxt, slot)
    pltpu.sync_copy(out_v, out_hbm.at[pl.ds(w * NPW, NPW)])


@functools.lru_cache(maxsize=1)
def _make_gather_max():
    return functools.partial(
        pl.kernel,
        mesh=plsc.VectorSubcoreMesh(core_axis_name="c", subcore_axis_name="s"),
        compiler_params=pltpu.CompilerParams(use_tc_tiling_on_sc=False),
        out_type=jax.ShapeDtypeStruct((NP, C), jnp.float32),
        scratch_types=[
            pltpu.VMEM((NCH, ECH), jnp.int32),
            pltpu.VMEM((NSLOT, ECH, CW), jnp.float32),
            pltpu.VMEM((NPW, C), jnp.float32),
            pltpu.SemaphoreType.DMA,
        ],
    )(_gather_max_sc_body)


def _gather_max(u, e):
    return _make_gather_max()(u, e)


PB = 1280


def _pool_body(h_ref, b_ref, wr_ref, br_ref, o_ref, acc_ref):
    blk = pl.program_id(0)
    nblk = pl.num_programs(0)
    h = h_ref[...]
    b = b_ref[...]
    neg = jnp.float32(-jnp.inf)

    @pl.when(blk == 0)
    def _():
        acc_ref[...] = jnp.full((16, C), neg, jnp.float32)

    g0 = jnp.min(b)
    g1 = jnp.minimum(jnp.max(b), 15)

    def seg(s, carry):
        m = jnp.max(jnp.where(b == s, h, neg), axis=0, keepdims=True)
        acc_ref[pl.ds(s, 1), :] = jnp.maximum(acc_ref[pl.ds(s, 1), :], m)
        return carry

    lax.fori_loop(g0, g1 + 1, seg, 0)

    @pl.when(blk == nblk - 1)
    def _():
        o_ref[...] = (jnp.dot(acc_ref[...], wr_ref[...],
                              preferred_element_type=jnp.float32)
                      + br_ref[...])


def _pool(h, batch_col, Wr, br):
    return pl.pallas_call(
        _pool_body,
        grid=(NP // PB,),
        in_specs=[
            pl.BlockSpec((PB, C), lambda i: (i, 0)),
            pl.BlockSpec((PB, 1), lambda i: (i, 0)),
            pl.BlockSpec((C, 6), lambda i: (0, 0)),
            pl.BlockSpec((1, 6), lambda i: (0, 0)),
        ],
        out_specs=pl.BlockSpec((16, 6), lambda i: (0, 0)),
        out_shape=jax.ShapeDtypeStruct((16, 6), jnp.float32),
        scratch_shapes=[pltpu.VMEM((16, C), jnp.float32)],
    )(h, batch_col, Wr, br)


def kernel(x, pos, batch, W1a, b1a, W1b, b1b, W2a, b2a, W2b, b2b,
           W3a, b3a, W3b, b3b, Wr, br):
    batch = batch.astype(jnp.int32)
    pos_p = jnp.pad(pos, ((0, NP - N), (0, 0)))
    x_p = jnp.pad(x, ((0, NP - N), (0, 0)))
    batch_p = jnp.pad(batch, (0, NP - N), constant_values=999)
    batch_col = batch_p[:, None]
    pos_tc = pos_p.T.reshape(3, NCB, CBW).transpose(1, 0, 2)
    batch_rc = batch_p.reshape(NCB, 1, CBW)

    seg = jnp.searchsorted(batch_p, jnp.arange(17, dtype=jnp.int32),
                           side='left').astype(jnp.int32)
    g0 = jnp.minimum(batch_p.reshape(NP // RB, RB)[:, 0], 16)
    g1 = jnp.minimum(batch_p.reshape(NP // RB, RB)[:, RB - 1], 16)
    c_lo = seg[g0] // CBW
    c_hi = (seg[jnp.minimum(g1 + 1, 16)] + CBW - 1) // CBW
    c_hi = jnp.maximum(c_hi, c_lo + 1)

    nbrs = _knn(pos_p, batch_col, pos_tc, batch_rc, c_lo, c_hi)
    e = nbrs.reshape(NW, NCH, ECH)

    h = jnp.concatenate([x_p, pos_p], axis=1)
    for Wa, ba, Wb, bb in ((W1a, b1a, W1b, b1b),
                           (W2a, b2a, W2b, b2b),
                           (W3a, b3a, W3b, b3b)):
        u = _mlp(h, Wa, ba[None, :], Wb, bb[None, :])
        h = _gather_max(u, e)

    return _pool(h, batch_col, Wr, br[None, :])


# speedup vs baseline: 1.0114x; 1.0114x over previous
import functools

import jax
import jax.numpy as jnp
from jax import lax
from jax.experimental import pallas as pl
from jax.experimental.pallas import tpu as pltpu
from jax.experimental.pallas import tpu_sc as plsc

N = 10000
NP = 10240
K = 6
C = 32
RB = 512
MB = 2560
NW = 32
NC = 2
NPW = NP // NW
EPW = NPW * K
NPCH = 16
ECH = NPCH * K
NCH = EPW // ECH
NSLOT = 20
CW = 32
BIG = 2 ** 30


CBW = 512
NCB = NP // CBW
HUGE = 1e30


def _knn_body(clo_ref, chi_ref, pos_blk, batch_blk, postc_ref, batchrc_ref,
              nbrs_ref):
    b = pl.program_id(0)
    c_lo = clo_ref[b]
    c_hi = chi_ref[b]
    pe = pos_blk[...]
    be = batch_blk[...]
    inf = jnp.float32(jnp.inf)
    rv0 = jnp.full((RB, 8), inf, jnp.float32)
    ri0 = jnp.full((RB, 8), BIG, jnp.int32)

    def chunk(cb, carry):
        rv, ri = carry
        pt = postc_ref[cb]
        br = batchrc_ref[cb]
        d2 = None
        for c in range(3):
            diff = pe[:, c:c + 1] - pt[c:c + 1, :]
            sq = diff * diff
            d2 = sq if d2 is None else d2 + sq
        d2 = jnp.where(be != br, jnp.float32(HUGE), d2)
        idxc = lax.broadcasted_iota(jnp.int32, (RB, CBW), 1) + cb * CBW
        full = jnp.concatenate([d2, rv], axis=1)
        fidx = jnp.concatenate([idxc, ri], axis=1)
        nrv, nri = [], []
        for _ in range(K):
            m = jnp.min(full, axis=1, keepdims=True)
            sel = jnp.where(full == m, fidx, BIG)
            j = jnp.min(sel, axis=1, keepdims=True)
            nrv.append(m)
            nri.append(j)
            full = jnp.where(fidx == j, inf, full)
        pad_v = jnp.full((RB, 8 - K), inf, jnp.float32)
        pad_i = jnp.full((RB, 8 - K), BIG, jnp.int32)
        return (jnp.concatenate(nrv + [pad_v], axis=1),
                jnp.concatenate(nri + [pad_i], axis=1))

    rv, ri = lax.fori_loop(c_lo, c_hi, chunk, (rv0, ri0))
    nbrs_ref[...] = jnp.minimum(ri[:, :K], NP - 1)


def _knn(pos_p, batch_col, pos_tc, batch_rc, c_lo, c_hi):
    return pl.pallas_call(
        _knn_body,
        grid=(NP // RB,),
        in_specs=[
            pl.BlockSpec(memory_space=pltpu.SMEM),
            pl.BlockSpec(memory_space=pltpu.SMEM),
            pl.BlockSpec((RB, 3), lambda i: (i, 0)),
            pl.BlockSpec((RB, 1), lambda i: (i, 0)),
            pl.BlockSpec((NCB, 3, CBW), lambda i: (0, 0, 0)),
            pl.BlockSpec((NCB, 1, CBW), lambda i: (0, 0, 0)),
        ],
        out_specs=pl.BlockSpec((RB, K), lambda i: (i, 0)),
        out_shape=jax.ShapeDtypeStruct((NP, K), jnp.int32),
    )(c_lo, c_hi, pos_p, batch_col, pos_tc, batch_rc)


def _mlp_body(h_ref, wa_ref, ba_ref, wb_ref, bb_ref, u_ref):
    a = jnp.dot(h_ref[...], wa_ref[...], preferred_element_type=jnp.float32)
    a = jnp.maximum(a + ba_ref[...], 0.0)
    u_ref[...] = (jnp.dot(a, wb_ref[...], preferred_element_type=jnp.float32)
                  + bb_ref[...])


def _mlp(h, Wa, ba, Wb, bb):
    cin = h.shape[1]
    return pl.pallas_call(
        _mlp_body,
        grid=(NP // MB,),
        in_specs=[
            pl.BlockSpec((MB, cin), lambda i: (i, 0)),
            pl.BlockSpec((cin, C), lambda i: (0, 0)),
            pl.BlockSpec((1, C), lambda i: (0, 0)),
            pl.BlockSpec((C, C), lambda i: (0, 0)),
            pl.BlockSpec((1, C), lambda i: (0, 0)),
        ],
        out_specs=pl.BlockSpec((MB, C), lambda i: (i, 0)),
        out_shape=jax.ShapeDtypeStruct((NP, C), jnp.float32),
    )(h, Wa, ba, Wb, bb)


def _gather_max_sc_body(u_hbm, e_hbm, out_hbm, idx_v, rows_v, out_v, sem):
    w = lax.axis_index("s") * NC + lax.axis_index("c")
    pltpu.sync_copy(e_hbm.at[w], idx_v)

    def fire(g, slot):
        return pltpu.async_copy(u_hbm.at[idx_v.at[g]], rows_v.at[slot], sem)

    cps = {}
    for g in range(min(NSLOT, NCH)):
        cps[g] = fire(g, g)
    for g in range(NCH):
        cps.pop(g).wait()
        slot = g % NSLOT

        def body(i, carry, slot=slot, g=g):
            base = i * K
            for c2 in range(C // 16):
                v = rows_v[slot, base, pl.ds(c2 * 16, 16)]
                for k in range(1, K):
                    v = jnp.maximum(v, rows_v[slot, base + k,
                                              pl.ds(c2 * 16, 16)])
                out_v[g * NPCH + i, pl.ds(c2 * 16, 16)] = jnp.maximum(v, 0.0)
            return carry

        lax.fori_loop(0, NPCH, body, 0)
        nxt = g + NSLOT
        if nxt < NCH:
            cps[nxt] = fire(nxt, slot)
    pltpu.sync_copy(out_v, out_hbm.at[pl.ds(w * NPW, NPW)])


@functools.lru_cache(maxsize=1)
def _make_gather_max():
    return functools.partial(
        pl.kernel,
        mesh=plsc.VectorSubcoreMesh(core_axis_name="c", subcore_axis_name="s"),
        compiler_params=pltpu.CompilerParams(use_tc_tiling_on_sc=False),
        out_type=jax.ShapeDtypeStruct((NP, C), jnp.float32),
        scratch_types=[
            pltpu.VMEM((NCH, ECH), jnp.int32),
            pltpu.VMEM((NSLOT, ECH, CW), jnp.float32),
            pltpu.VMEM((NPW, C), jnp.float32),
            pltpu.SemaphoreType.DMA,
        ],
    )(_gather_max_sc_body)


def _gather_max(u, e):
    return _make_gather_max()(u, e)


PB = 1280


def _pool_body(h_ref, b_ref, wr_ref, br_ref, o_ref, acc_ref):
    blk = pl.program_id(0)
    nblk = pl.num_programs(0)
    h = h_ref[...]
    b = b_ref[...]
    neg = jnp.float32(-jnp.inf)

    @pl.when(blk == 0)
    def _():
        acc_ref[...] = jnp.full((16, C), neg, jnp.float32)

    g0 = jnp.min(b)
    g1 = jnp.minimum(jnp.max(b), 15)

    def seg(s, carry):
        m = jnp.max(jnp.where(b == s, h, neg), axis=0, keepdims=True)
        acc_ref[pl.ds(s, 1), :] = jnp.maximum(acc_ref[pl.ds(s, 1), :], m)
        return carry

    lax.fori_loop(g0, g1 + 1, seg, 0)

    @pl.when(blk == nblk - 1)
    def _():
        o_ref[...] = (jnp.dot(acc_ref[...], wr_ref[...],
                              preferred_element_type=jnp.float32)
                      + br_ref[...])


def _pool(h, batch_col, Wr, br):
    return pl.pallas_call(
        _pool_body,
        grid=(NP // PB,),
        in_specs=[
            pl.BlockSpec((PB, C), lambda i: (i, 0)),
            pl.BlockSpec((PB, 1), lambda i: (i, 0)),
            pl.BlockSpec((C, 6), lambda i: (0, 0)),
            pl.BlockSpec((1, 6), lambda i: (0, 0)),
        ],
        out_specs=pl.BlockSpec((16, 6), lambda i: (0, 0)),
        out_shape=jax.ShapeDtypeStruct((16, 6), jnp.float32),
        scratch_shapes=[pltpu.VMEM((16, C), jnp.float32)],
    )(h, batch_col, Wr, br)


def kernel(x, pos, batch, W1a, b1a, W1b, b1b, W2a, b2a, W2b, b2b,
           W3a, b3a, W3b, b3b, Wr, br):
    batch = batch.astype(jnp.int32)
    pos_p = jnp.pad(pos, ((0, NP - N), (0, 0)))
    x_p = jnp.pad(x, ((0, NP - N), (0, 0)))
    batch_p = jnp.pad(batch, (0, NP - N), constant_values=999)
    batch_col = batch_p[:, None]
    pos_tc = pos_p.T.reshape(3, NCB, CBW).transpose(1, 0, 2)
    batch_rc = batch_p.reshape(NCB, 1, CBW)

    seg = jnp.searchsorted(batch_p, jnp.arange(17, dtype=jnp.int32),
                           side='left').astype(jnp.int32)
    g0 = jnp.minimum(batch_p.reshape(NP // RB, RB)[:, 0], 16)
    g1 = jnp.minimum(batch_p.reshape(NP // RB, RB)[:, RB - 1], 16)
    c_lo = seg[g0] // CBW
    c_hi = (seg[jnp.minimum(g1 + 1, 16)] + CBW - 1) // CBW
    c_hi = jnp.maximum(c_hi, c_lo + 1)

    nbrs = _knn(pos_p, batch_col, pos_tc, batch_rc, c_lo, c_hi)
    e = nbrs.reshape(NW, NCH, ECH)

    h = jnp.concatenate([x_p, pos_p], axis=1)
    for Wa, ba, Wb, bb in ((W1a, b1a, W1b, b1b),
                           (W2a, b2a, W2b, b2b),
                           (W3a, b3a, W3b, b3b)):
        u = _mlp(h, Wa, ba[None, :], Wb, bb[None, :])
        h = _gather_max(u, e)

    return _pool(h, batch_col, Wr, br[None, :])
